# Initial kernel scaffold; baseline (speedup 1.0000x reference)
import jax
import jax.numpy as jnp
from jax.experimental import pallas as pl


def kernel(x, edge_index, edge_attr, W1_l, W1_r, b1, W2_l, W2_r, b2):
    raise NotImplementedError("write your pallas kernel here")


# speedup vs baseline: 4.8260x; 4.8260x over previous
import functools

import jax
import jax.numpy as jnp
from jax import lax
from jax.experimental import pallas as pl
from jax.experimental.pallas import tpu as pltpu
from jax.experimental.pallas import tpu_sc as plsc

N_NODES = 10000
N_EDGES = 320000
D = 128

NC = 2
NS = 16
NW = NC * NS
EPW = N_EDGES // NW
CH = 80
ITERS = EPW // CH
RPT = 624
TAIL = N_NODES - NS * RPT
ZR = 16

_mesh = plsc.VectorSubcoreMesh(core_axis_name="c", subcore_axis_name="s")


def _fill_zero(ref, nrows):
    zv = jnp.zeros((16,), jnp.float32)

    def fill(i, _):
        for j in range(D // 16):
            ref[i, pl.ds(j * 16, 16)] = zv
        return 0
    lax.fori_loop(0, nrows, fill, 0)


def _zero_acc(zbuf, acc, s):
    def zero_blk(k, _):
        pltpu.sync_copy(zbuf, acc.at[pl.ds(s * RPT + k * ZR, ZR)])
        return 0
    lax.fori_loop(0, RPT // ZR, zero_blk, 0)

    @pl.when(s == NS - 1)
    def _zero_tail():
        pltpu.sync_copy(zbuf, acc.at[pl.ds(NS * RPT, TAIL)])


def _writeback(acc, out_hbm, c, s):
    pltpu.sync_copy(acc.at[pl.ds(s * RPT, RPT)],
                    out_hbm.at[c, pl.ds(s * RPT, RPT)])

    @pl.when(s == NS - 1)
    def _write_tail():
        pltpu.sync_copy(acc.at[pl.ds(NS * RPT, TAIL)],
                        out_hbm.at[c, pl.ds(NS * RPT, TAIL)])


@functools.partial(
    pl.kernel, mesh=_mesh,
    out_type=jax.ShapeDtypeStruct((NC, N_NODES, D), jnp.float32),
    scratch_types=[
        pltpu.VMEM((1, CH), jnp.int32),
        pltpu.VMEM((1, CH), jnp.int32),
        pltpu.VMEM((CH, D), jnp.float32),
        pltpu.VMEM((ZR, D), jnp.float32),
        pltpu.SemaphoreType.DMA,
        pltpu.VMEM_SHARED((N_NODES, D), jnp.float32),
    ],
)
def _sc_agg(feat, src, dst, agg_hbm, idx_s, idx_d, rows, zbuf, sem, acc):
    c = lax.axis_index("c")
    s = lax.axis_index("s")
    wid = s * NC + c

    _fill_zero(zbuf, ZR)
    _zero_acc(zbuf, acc, s)
    plsc.subcore_barrier()

    def step(it, _):
        base = wid * EPW + it * CH
        pltpu.sync_copy(src.at[pl.ds(base, CH)], idx_s.at[0])
        pltpu.sync_copy(dst.at[pl.ds(base, CH)], idx_d.at[0])
        pltpu.async_copy(feat.at[idx_s.at[0]], rows, sem).wait()
        pltpu.sync_copy(rows, acc.at[idx_d.at[0]], add=True)
        return 0
    lax.fori_loop(0, ITERS, step, 0)
    plsc.subcore_barrier()

    _writeback(acc, agg_hbm, c, s)


@functools.partial(
    pl.kernel, mesh=_mesh,
    out_type=jax.ShapeDtypeStruct((NC, N_NODES, D), jnp.float32),
    scratch_types=[
        pltpu.VMEM((1, CH), jnp.int32),
        pltpu.VMEM((CH, D), jnp.float32),
        pltpu.VMEM((ZR, D), jnp.float32),
        pltpu.VMEM_SHARED((N_NODES, D), jnp.float32),
    ],
)
def _sc_deg(dst, deg_hbm, idx_d, ones, zbuf, acc):
    c = lax.axis_index("c")
    s = lax.axis_index("s")
    wid = s * NC + c

    zv = jnp.zeros((16,), jnp.float32)
    iota = lax.iota(jnp.int32, 16)
    onev = jnp.where(iota == 0, 1.0, 0.0).astype(jnp.float32)

    _fill_zero(zbuf, ZR)

    def fill_ones(i, _):
        ones[i, pl.ds(0, 16)] = onev
        for j in range(1, D // 16):
            ones[i, pl.ds(j * 16, 16)] = zv
        return 0
    lax.fori_loop(0, CH, fill_ones, 0)

    _zero_acc(zbuf, acc, s)
    plsc.subcore_barrier()

    def step(it, _):
        base = wid * EPW + it * CH
        pltpu.sync_copy(dst.at[pl.ds(base, CH)], idx_d.at[0])
        pltpu.sync_copy(ones, acc.at[idx_d.at[0]], add=True)
        return 0
    lax.fori_loop(0, ITERS, step, 0)
    plsc.subcore_barrier()

    _writeback(acc, deg_hbm, c, s)


ROWS_BLK = 1000


def _tc_body(apply_relu, p_ref, dg_ref, x_ref, wl_ref, wr_ref, b_ref, o_ref):
    agg = p_ref[0] + p_ref[1]
    deg = dg_ref[0, :, 0:1] + dg_ref[1, :, 0:1]
    mean = agg / jnp.maximum(deg, 1.0)
    h = jnp.dot(mean, wl_ref[...], preferred_element_type=jnp.float32)
    h = h + jnp.dot(x_ref[...], wr_ref[...], preferred_element_type=jnp.float32)
    h = h + b_ref[...]
    if apply_relu:
        h = jnp.maximum(h, 0.0)
    o_ref[...] = h


def _tc_combine(p, dg, x, wl, wr, b, apply_relu):
    grid = (N_NODES // ROWS_BLK,)
    return pl.pallas_call(
        functools.partial(_tc_body, apply_relu),
        grid=grid,
        in_specs=[
            pl.BlockSpec((NC, ROWS_BLK, D), lambda i: (0, i, 0)),
            pl.BlockSpec((NC, ROWS_BLK, D), lambda i: (0, i, 0)),
            pl.BlockSpec((ROWS_BLK, D), lambda i: (i, 0)),
            pl.BlockSpec((D, D), lambda i: (0, 0)),
            pl.BlockSpec((D, D), lambda i: (0, 0)),
            pl.BlockSpec((1, D), lambda i: (0, 0)),
        ],
        out_specs=pl.BlockSpec((ROWS_BLK, D), lambda i: (i, 0)),
        out_shape=jax.ShapeDtypeStruct((N_NODES, D), jnp.float32),
    )(p, dg, x, wl, wr, b)


def kernel(x, edge_index, edge_attr, W1_l, W1_r, b1, W2_l, W2_r, b2):
    src = edge_index[0].astype(jnp.int32)
    dst = edge_index[1].astype(jnp.int32)
    agg1 = _sc_agg(x, src, dst)
    degp = _sc_deg(dst)
    h = _tc_combine(agg1, degp, x, W1_l, W1_r, b1.reshape(1, D), True)
    agg2 = _sc_agg(h, src, dst)
    out = _tc_combine(agg2, degp, h, W2_l, W2_r, b2.reshape(1, D), False)
    return out


# speedup vs baseline: 5.9728x; 1.2376x over previous
import functools

import jax
import jax.numpy as jnp
from jax import lax
from jax.experimental import pallas as pl
from jax.experimental.pallas import tpu as pltpu
from jax.experimental.pallas import tpu_sc as plsc

N_NODES = 10000
N_EDGES = 320000
D = 128

NC = 2
NS = 16
NW = NC * NS
EPW = N_EDGES // NW
CH = 80
ITERS = EPW // CH
RPT = 624
TAIL = N_NODES - NS * RPT
ZR = 16

_mesh = plsc.VectorSubcoreMesh(core_axis_name="c", subcore_axis_name="s")


def _fill_zero(ref, nrows):
    zv = jnp.zeros((16,), jnp.float32)

    def fill(i, _):
        for j in range(D // 16):
            ref[i, pl.ds(j * 16, 16)] = zv
        return 0
    lax.fori_loop(0, nrows, fill, 0)


def _zero_acc(zbuf, acc, s):
    def zero_blk(k, _):
        pltpu.sync_copy(zbuf, acc.at[pl.ds(s * RPT + k * ZR, ZR)])
        return 0
    lax.fori_loop(0, RPT // ZR, zero_blk, 0)

    @pl.when(s == NS - 1)
    def _zero_tail():
        pltpu.sync_copy(zbuf, acc.at[pl.ds(NS * RPT, TAIL)])


def _writeback(acc, out_hbm, c, s):
    pltpu.sync_copy(acc.at[pl.ds(s * RPT, RPT)],
                    out_hbm.at[c, pl.ds(s * RPT, RPT)])

    @pl.when(s == NS - 1)
    def _write_tail():
        pltpu.sync_copy(acc.at[pl.ds(NS * RPT, TAIL)],
                        out_hbm.at[c, pl.ds(NS * RPT, TAIL)])


@functools.partial(
    pl.kernel, mesh=_mesh,
    out_type=jax.ShapeDtypeStruct((NC, N_NODES, D), jnp.float32),
    scratch_types=[
        pltpu.VMEM((2, CH), jnp.int32),
        pltpu.VMEM((2, CH), jnp.int32),
        pltpu.VMEM((CH, D), jnp.float32),
        pltpu.VMEM((CH, D), jnp.float32),
        pltpu.VMEM((ZR, D), jnp.float32),
        pltpu.SemaphoreType.DMA,
        pltpu.SemaphoreType.DMA,
        pltpu.SemaphoreType.DMA,
        pltpu.SemaphoreType.DMA,
        pltpu.VMEM_SHARED((N_NODES, D), jnp.float32),
    ],
)
def _sc_agg(feat, src, dst, agg_hbm, idx_s, idx_d, rows0, rows1, zbuf,
            sg0, sg1, ss0, ss1, acc):
    c = lax.axis_index("c")
    s = lax.axis_index("s")
    wid = s * NC + c
    e0 = wid * EPW

    _fill_zero(zbuf, ZR)
    _zero_acc(zbuf, acc, s)
    plsc.subcore_barrier()

    def load_idx(slot, base):
        pltpu.sync_copy(src.at[pl.ds(base, CH)], idx_s.at[slot])
        pltpu.sync_copy(dst.at[pl.ds(base, CH)], idx_d.at[slot])

    load_idx(0, e0)
    pltpu.async_copy(feat.at[idx_s.at[0]], rows0, sg0)

    def pair(k, _):
        pltpu.make_async_copy(feat.at[idx_s.at[0]], rows0, sg0).wait()

        @pl.when(k > 0)
        def _():
            pltpu.make_async_copy(rows1, acc.at[idx_d.at[1]], ss1).wait()
        load_idx(1, e0 + (2 * k + 1) * CH)
        pltpu.async_copy(feat.at[idx_s.at[1]], rows1, sg1)
        pltpu.async_copy(rows0, acc.at[idx_d.at[0]], ss0, add=True)
        pltpu.make_async_copy(feat.at[idx_s.at[1]], rows1, sg1).wait()
        pltpu.make_async_copy(rows0, acc.at[idx_d.at[0]], ss0).wait()
        pltpu.async_copy(rows1, acc.at[idx_d.at[1]], ss1, add=True)
        load_idx(0, e0 + (2 * k + 2) * CH)
        pltpu.async_copy(feat.at[idx_s.at[0]], rows0, sg0)
        return 0
    lax.fori_loop(0, ITERS // 2, pair, 0)

    pltpu.make_async_copy(feat.at[idx_s.at[0]], rows0, sg0).wait()
    pltpu.make_async_copy(rows1, acc.at[idx_d.at[1]], ss1).wait()
    pltpu.sync_copy(rows0, acc.at[idx_d.at[0]], add=True)
    plsc.subcore_barrier()

    _writeback(acc, agg_hbm, c, s)


@functools.partial(
    pl.kernel, mesh=_mesh,
    out_type=jax.ShapeDtypeStruct((NC, N_NODES, D), jnp.float32),
    scratch_types=[
        pltpu.VMEM((2, CH), jnp.int32),
        pltpu.VMEM((CH, D), jnp.float32),
        pltpu.VMEM((ZR, D), jnp.float32),
        pltpu.SemaphoreType.DMA,
        pltpu.SemaphoreType.DMA,
        pltpu.VMEM_SHARED((N_NODES, D), jnp.float32),
    ],
)
def _sc_deg(dst, deg_hbm, idx_d, ones, zbuf, ss0, ss1, acc):
    c = lax.axis_index("c")
    s = lax.axis_index("s")
    wid = s * NC + c
    e0 = wid * EPW

    zv = jnp.zeros((16,), jnp.float32)
    iota = lax.iota(jnp.int32, 16)
    onev = jnp.where(iota == 0, 1.0, 0.0).astype(jnp.float32)

    _fill_zero(zbuf, ZR)

    def fill_ones(i, _):
        ones[i, pl.ds(0, 16)] = onev
        for j in range(1, D // 16):
            ones[i, pl.ds(j * 16, 16)] = zv
        return 0
    lax.fori_loop(0, CH, fill_ones, 0)

    _zero_acc(zbuf, acc, s)
    plsc.subcore_barrier()

    pltpu.sync_copy(dst.at[pl.ds(e0, CH)], idx_d.at[0])

    def pair(k, _):
        pltpu.async_copy(ones, acc.at[idx_d.at[0]], ss0, add=True)

        @pl.when(k > 0)
        def _():
            pltpu.make_async_copy(ones, acc.at[idx_d.at[1]], ss1).wait()
        pltpu.sync_copy(dst.at[pl.ds(e0 + (2 * k + 1) * CH, CH)],
                        idx_d.at[1])
        pltpu.async_copy(ones, acc.at[idx_d.at[1]], ss1, add=True)
        pltpu.make_async_copy(ones, acc.at[idx_d.at[0]], ss0).wait()
        pltpu.sync_copy(dst.at[pl.ds(e0 + (2 * k + 2) * CH, CH)],
                        idx_d.at[0])
        return 0
    lax.fori_loop(0, ITERS // 2, pair, 0)

    pltpu.make_async_copy(ones, acc.at[idx_d.at[1]], ss1).wait()
    pltpu.sync_copy(ones, acc.at[idx_d.at[0]], add=True)
    plsc.subcore_barrier()

    _writeback(acc, deg_hbm, c, s)


ROWS_BLK = 1000


def _tc_body(apply_relu, p_ref, dg_ref, x_ref, wl_ref, wr_ref, b_ref, o_ref):
    agg = p_ref[0] + p_ref[1]
    deg = dg_ref[0, :, 0:1] + dg_ref[1, :, 0:1]
    mean = agg / jnp.maximum(deg, 1.0)
    h = jnp.dot(mean, wl_ref[...], preferred_element_type=jnp.float32)
    h = h + jnp.dot(x_ref[...], wr_ref[...], preferred_element_type=jnp.float32)
    h = h + b_ref[...]
    if apply_relu:
        h = jnp.maximum(h, 0.0)
    o_ref[...] = h


def _tc_combine(p, dg, x, wl, wr, b, apply_relu):
    grid = (N_NODES // ROWS_BLK,)
    return pl.pallas_call(
        functools.partial(_tc_body, apply_relu),
        grid=grid,
        in_specs=[
            pl.BlockSpec((NC, ROWS_BLK, D), lambda i: (0, i, 0)),
            pl.BlockSpec((NC, ROWS_BLK, D), lambda i: (0, i, 0)),
            pl.BlockSpec((ROWS_BLK, D), lambda i: (i, 0)),
            pl.BlockSpec((D, D), lambda i: (0, 0)),
            pl.BlockSpec((D, D), lambda i: (0, 0)),
            pl.BlockSpec((1, D), lambda i: (0, 0)),
        ],
        out_specs=pl.BlockSpec((ROWS_BLK, D), lambda i: (i, 0)),
        out_shape=jax.ShapeDtypeStruct((N_NODES, D), jnp.float32),
    )(p, dg, x, wl, wr, b)


def kernel(x, edge_index, edge_attr, W1_l, W1_r, b1, W2_l, W2_r, b2):
    src = edge_index[0].astype(jnp.int32)
    dst = edge_index[1].astype(jnp.int32)
    agg1 = _sc_agg(x, src, dst)
    degp = _sc_deg(dst)
    h = _tc_combine(agg1, degp, x, W1_l, W1_r, b1.reshape(1, D), True)
    agg2 = _sc_agg(h, src, dst)
    out = _tc_combine(agg2, degp, h, W2_l, W2_r, b2.reshape(1, D), False)
    return out


# speedup vs baseline: 7.4885x; 1.2538x over previous
import functools

import jax
import jax.numpy as jnp
from jax import lax
from jax.experimental import pallas as pl
from jax.experimental.pallas import tpu as pltpu
from jax.experimental.pallas import tpu_sc as plsc

N_NODES = 10000
N_EDGES = 320000
D = 128

NC = 2
NS = 16
NW = NC * NS
CH = 128
NCHUNK = N_EDGES // CH
CPW = NCHUNK // NW
NEXTRA = NCHUNK - CPW * NW
PAIRS = CPW // 2
RPT = 624
TAIL = N_NODES - NS * RPT
ZR = 16

_mesh = plsc.VectorSubcoreMesh(core_axis_name="c", subcore_axis_name="s")


def _fill_zero(ref, nrows):
    zv = jnp.zeros((16,), jnp.float32)

    def fill(i, _):
        for j in range(D // 16):
            ref[i, pl.ds(j * 16, 16)] = zv
        return 0
    lax.fori_loop(0, nrows, fill, 0)


def _zero_acc(zbuf, acc, s):
    def zero_blk(k, _):
        pltpu.sync_copy(zbuf, acc.at[pl.ds(s * RPT + k * ZR, ZR)])
        return 0
    lax.fori_loop(0, RPT // ZR, zero_blk, 0)

    @pl.when(s == NS - 1)
    def _zero_tail():
        pltpu.sync_copy(zbuf, acc.at[pl.ds(NS * RPT, TAIL)])


def _writeback(acc, out_hbm, c, s):
    pltpu.sync_copy(acc.at[pl.ds(s * RPT, RPT)],
                    out_hbm.at[c, pl.ds(s * RPT, RPT)])

    @pl.when(s == NS - 1)
    def _write_tail():
        pltpu.sync_copy(acc.at[pl.ds(NS * RPT, TAIL)],
                        out_hbm.at[c, pl.ds(NS * RPT, TAIL)])


def _chunk0(wid):
    return CPW * wid + jnp.minimum(wid, NEXTRA)


def _clamp(cidx):
    return jnp.minimum(cidx, NCHUNK - 1)


@functools.partial(
    pl.kernel, mesh=_mesh,
    out_type=jax.ShapeDtypeStruct((NC, N_NODES, D), jnp.float32),
    scratch_types=[
        pltpu.VMEM((2, 1, CH), jnp.int32),
        pltpu.VMEM((2, 1, CH), jnp.int32),
        pltpu.VMEM((CH, D), jnp.float32),
        pltpu.VMEM((CH, D), jnp.float32),
        pltpu.VMEM((ZR, D), jnp.float32),
        pltpu.SemaphoreType.DMA,
        pltpu.SemaphoreType.DMA,
        pltpu.SemaphoreType.DMA,
        pltpu.SemaphoreType.DMA,
        pltpu.VMEM_SHARED((N_NODES, D), jnp.float32),
    ],
)
def _sc_agg(feat, src3, dst3, agg_hbm, idx_s, idx_d, rows0, rows1, zbuf,
            sg0, sg1, ss0, ss1, acc):
    c = lax.axis_index("c")
    s = lax.axis_index("s")
    wid = s * NC + c
    c0 = _chunk0(wid)

    _fill_zero(zbuf, ZR)
    _zero_acc(zbuf, acc, s)
    plsc.subcore_barrier()

    def load_idx(slot, cidx):
        cidx = _clamp(cidx)
        pltpu.sync_copy(src3.at[cidx], idx_s.at[slot])
        pltpu.sync_copy(dst3.at[cidx], idx_d.at[slot])

    load_idx(0, c0)
    pltpu.async_copy(feat.at[idx_s.at[0, 0]], rows0, sg0)

    def pair(k, _):
        pltpu.make_async_copy(feat.at[idx_s.at[0, 0]], rows0, sg0).wait()

        @pl.when(k > 0)
        def _():
            pltpu.make_async_copy(rows1, acc.at[idx_d.at[1, 0]], ss1).wait()
        load_idx(1, c0 + 2 * k + 1)
        pltpu.async_copy(feat.at[idx_s.at[1, 0]], rows1, sg1)
        pltpu.async_copy(rows0, acc.at[idx_d.at[0, 0]], ss0, add=True)
        pltpu.make_async_copy(feat.at[idx_s.at[1, 0]], rows1, sg1).wait()
        pltpu.make_async_copy(rows0, acc.at[idx_d.at[0, 0]], ss0).wait()
        pltpu.async_copy(rows1, acc.at[idx_d.at[1, 0]], ss1, add=True)
        load_idx(0, c0 + 2 * k + 2)
        pltpu.async_copy(feat.at[idx_s.at[0, 0]], rows0, sg0)
        return 0
    lax.fori_loop(0, PAIRS, pair, 0)

    pltpu.make_async_copy(feat.at[idx_s.at[0, 0]], rows0, sg0).wait()
    pltpu.make_async_copy(rows1, acc.at[idx_d.at[1, 0]], ss1).wait()

    @pl.when(wid < NEXTRA)
    def _last():
        pltpu.sync_copy(rows0, acc.at[idx_d.at[0, 0]], add=True)
    plsc.subcore_barrier()

    _writeback(acc, agg_hbm, c, s)


@functools.partial(
    pl.kernel, mesh=_mesh,
    out_type=jax.ShapeDtypeStruct((NC, N_NODES, D), jnp.float32),
    scratch_types=[
        pltpu.VMEM((2, 1, CH), jnp.int32),
        pltpu.VMEM((CH, D), jnp.float32),
        pltpu.VMEM((ZR, D), jnp.float32),
        pltpu.SemaphoreType.DMA,
        pltpu.SemaphoreType.DMA,
        pltpu.VMEM_SHARED((N_NODES, D), jnp.float32),
    ],
)
def _sc_deg(dst3, deg_hbm, idx_d, ones, zbuf, ss0, ss1, acc):
    c = lax.axis_index("c")
    s = lax.axis_index("s")
    wid = s * NC + c
    c0 = _chunk0(wid)

    zv = jnp.zeros((16,), jnp.float32)
    iota = lax.iota(jnp.int32, 16)
    onev = jnp.where(iota == 0, 1.0, 0.0).astype(jnp.float32)

    _fill_zero(zbuf, ZR)

    def fill_ones(i, _):
        ones[i, pl.ds(0, 16)] = onev
        for j in range(1, D // 16):
            ones[i, pl.ds(j * 16, 16)] = zv
        return 0
    lax.fori_loop(0, CH, fill_ones, 0)

    _zero_acc(zbuf, acc, s)
    plsc.subcore_barrier()

    pltpu.sync_copy(dst3.at[c0], idx_d.at[0])

    def pair(k, _):
        pltpu.async_copy(ones, acc.at[idx_d.at[0, 0]], ss0, add=True)

        @pl.when(k > 0)
        def _():
            pltpu.make_async_copy(ones, acc.at[idx_d.at[1, 0]], ss1).wait()
        pltpu.sync_copy(dst3.at[_clamp(c0 + 2 * k + 1)], idx_d.at[1])
        pltpu.async_copy(ones, acc.at[idx_d.at[1, 0]], ss1, add=True)
        pltpu.make_async_copy(ones, acc.at[idx_d.at[0, 0]], ss0).wait()
        pltpu.sync_copy(dst3.at[_clamp(c0 + 2 * k + 2)], idx_d.at[0])
        return 0
    lax.fori_loop(0, PAIRS, pair, 0)

    pltpu.make_async_copy(ones, acc.at[idx_d.at[1, 0]], ss1).wait()

    @pl.when(wid < NEXTRA)
    def _last():
        pltpu.sync_copy(ones, acc.at[idx_d.at[0, 0]], add=True)
    plsc.subcore_barrier()

    _writeback(acc, deg_hbm, c, s)


ROWS_BLK = 1000


def _tc_body(apply_relu, p_ref, dg_ref, x_ref, wl_ref, wr_ref, b_ref, o_ref):
    agg = p_ref[0] + p_ref[1]
    deg = dg_ref[0, :, 0:1] + dg_ref[1, :, 0:1]
    mean = agg / jnp.maximum(deg, 1.0)
    h = jnp.dot(mean, wl_ref[...], preferred_element_type=jnp.float32)
    h = h + jnp.dot(x_ref[...], wr_ref[...], preferred_element_type=jnp.float32)
    h = h + b_ref[...]
    if apply_relu:
        h = jnp.maximum(h, 0.0)
    o_ref[...] = h


def _tc_combine(p, dg, x, wl, wr, b, apply_relu):
    grid = (N_NODES // ROWS_BLK,)
    return pl.pallas_call(
        functools.partial(_tc_body, apply_relu),
        grid=grid,
        in_specs=[
            pl.BlockSpec((NC, ROWS_BLK, D), lambda i: (0, i, 0)),
            pl.BlockSpec((NC, ROWS_BLK, D), lambda i: (0, i, 0)),
            pl.BlockSpec((ROWS_BLK, D), lambda i: (i, 0)),
            pl.BlockSpec((D, D), lambda i: (0, 0)),
            pl.BlockSpec((D, D), lambda i: (0, 0)),
            pl.BlockSpec((1, D), lambda i: (0, 0)),
        ],
        out_specs=pl.BlockSpec((ROWS_BLK, D), lambda i: (i, 0)),
        out_shape=jax.ShapeDtypeStruct((N_NODES, D), jnp.float32),
    )(p, dg, x, wl, wr, b)


def kernel(x, edge_index, edge_attr, W1_l, W1_r, b1, W2_l, W2_r, b2):
    src3 = edge_index[0].astype(jnp.int32).reshape(NCHUNK, 1, CH)
    dst3 = edge_index[1].astype(jnp.int32).reshape(NCHUNK, 1, CH)
    agg1 = _sc_agg(x, src3, dst3)
    degp = _sc_deg(dst3)
    h = _tc_combine(agg1, degp, x, W1_l, W1_r, b1.reshape(1, D), True)
    agg2 = _sc_agg(h, src3, dst3)
    out = _tc_combine(agg2, degp, h, W2_l, W2_r, b2.reshape(1, D), False)
    return out


# speedup vs baseline: 8.6475x; 1.1548x over previous
import functools

import jax
import jax.numpy as jnp
from jax import lax
from jax.experimental import pallas as pl
from jax.experimental.pallas import tpu as pltpu
from jax.experimental.pallas import tpu_sc as plsc

N_NODES = 10000
N_EDGES = 320000
D = 128

NC = 2
NS = 16
NW = NC * NS
CH = 128
NCHUNK = N_EDGES // CH
CPW = NCHUNK // NW
NEXTRA = NCHUNK - CPW * NW
PAIRS = CPW // 2
RPT = 624
TAIL = N_NODES - NS * RPT
ZR = 16

_mesh = plsc.VectorSubcoreMesh(core_axis_name="c", subcore_axis_name="s")


def _fill_zero(ref, nrows):
    zv = jnp.zeros((16,), jnp.float32)

    def fill(i, _):
        for j in range(D // 16):
            ref[i, pl.ds(j * 16, 16)] = zv
        return 0
    lax.fori_loop(0, nrows, fill, 0)


def _zero_acc(zbuf, acc, s):
    def zero_blk(k, _):
        pltpu.sync_copy(zbuf, acc.at[pl.ds(s * RPT + k * ZR, ZR)])
        return 0
    lax.fori_loop(0, RPT // ZR, zero_blk, 0)

    @pl.when(s == NS - 1)
    def _zero_tail():
        pltpu.sync_copy(zbuf, acc.at[pl.ds(NS * RPT, TAIL)])


def _writeback(acc, out_hbm, c, s):
    pltpu.sync_copy(acc.at[pl.ds(s * RPT, RPT)],
                    out_hbm.at[c, pl.ds(s * RPT, RPT)])

    @pl.when(s == NS - 1)
    def _write_tail():
        pltpu.sync_copy(acc.at[pl.ds(NS * RPT, TAIL)],
                        out_hbm.at[c, pl.ds(NS * RPT, TAIL)])


def _chunk0(wid):
    return CPW * wid + jnp.minimum(wid, NEXTRA)


def _clamp(cidx):
    return jnp.minimum(cidx, NCHUNK - 1)


@functools.partial(
    pl.kernel, mesh=_mesh,
    out_type=jax.ShapeDtypeStruct((NC, N_NODES, D), jnp.float32),
    scratch_types=[
        pltpu.VMEM((2, 1, CH), jnp.int32),
        pltpu.VMEM((2, 1, CH), jnp.int32),
        pltpu.VMEM((CH, D), jnp.float32),
        pltpu.VMEM((CH, D), jnp.float32),
        pltpu.VMEM((ZR, D), jnp.float32),
        pltpu.SemaphoreType.DMA,
        pltpu.SemaphoreType.DMA,
        pltpu.SemaphoreType.DMA,
        pltpu.SemaphoreType.DMA,
        pltpu.SemaphoreType.DMA,
        pltpu.SemaphoreType.DMA,
        pltpu.VMEM_SHARED((N_NODES, D), jnp.float32),
    ],
)
def _sc_agg(feat, src3, dst3, agg_hbm, idx_s, idx_d, rows0, rows1, zbuf,
            sg0, sg1, ss0, ss1, si0, si1, acc):
    c = lax.axis_index("c")
    s = lax.axis_index("s")
    wid = s * NC + c
    c0 = _chunk0(wid)

    _fill_zero(zbuf, ZR)
    _zero_acc(zbuf, acc, s)
    plsc.subcore_barrier()

    pltpu.sync_copy(src3.at[c0], idx_s.at[0])
    pltpu.sync_copy(dst3.at[c0], idx_d.at[0])
    pltpu.async_copy(feat.at[idx_s.at[0, 0]], rows0, sg0)

    def pair(k, _):
        cb = _clamp(c0 + 2 * k + 1)
        ca2 = _clamp(c0 + 2 * k + 2)
        pltpu.make_async_copy(feat.at[idx_s.at[0, 0]], rows0, sg0).wait()
        pltpu.async_copy(src3.at[cb], idx_s.at[1], si1)

        @pl.when(k > 0)
        def _():
            pltpu.make_async_copy(rows1, acc.at[idx_d.at[1, 0]], ss1).wait()
        pltpu.async_copy(dst3.at[cb], idx_d.at[1], si1)
        pltpu.make_async_copy(src3.at[cb], idx_s.at[1], si1).wait()
        pltpu.make_async_copy(dst3.at[cb], idx_d.at[1], si1).wait()
        pltpu.async_copy(feat.at[idx_s.at[1, 0]], rows1, sg1)
        pltpu.async_copy(rows0, acc.at[idx_d.at[0, 0]], ss0, add=True)
        pltpu.async_copy(src3.at[ca2], idx_s.at[0], si0)
        pltpu.make_async_copy(feat.at[idx_s.at[1, 0]], rows1, sg1).wait()
        pltpu.make_async_copy(rows0, acc.at[idx_d.at[0, 0]], ss0).wait()
        pltpu.async_copy(dst3.at[ca2], idx_d.at[0], si0)
        pltpu.async_copy(rows1, acc.at[idx_d.at[1, 0]], ss1, add=True)
        pltpu.make_async_copy(src3.at[ca2], idx_s.at[0], si0).wait()
        pltpu.make_async_copy(dst3.at[ca2], idx_d.at[0], si0).wait()
        pltpu.async_copy(feat.at[idx_s.at[0, 0]], rows0, sg0)
        return 0
    lax.fori_loop(0, PAIRS, pair, 0)

    pltpu.make_async_copy(feat.at[idx_s.at[0, 0]], rows0, sg0).wait()
    pltpu.make_async_copy(rows1, acc.at[idx_d.at[1, 0]], ss1).wait()

    @pl.when(wid < NEXTRA)
    def _last():
        pltpu.sync_copy(rows0, acc.at[idx_d.at[0, 0]], add=True)
    plsc.subcore_barrier()

    _writeback(acc, agg_hbm, c, s)


@functools.partial(
    pl.kernel, mesh=_mesh,
    out_type=jax.ShapeDtypeStruct((NC, N_NODES, D), jnp.float32),
    scratch_types=[
        pltpu.VMEM((2, 1, CH), jnp.int32),
        pltpu.VMEM((CH, D), jnp.float32),
        pltpu.VMEM((ZR, D), jnp.float32),
        pltpu.SemaphoreType.DMA,
        pltpu.SemaphoreType.DMA,
        pltpu.VMEM_SHARED((N_NODES, D), jnp.float32),
    ],
)
def _sc_deg(dst3, deg_hbm, idx_d, ones, zbuf, ss0, ss1, acc):
    c = lax.axis_index("c")
    s = lax.axis_index("s")
    wid = s * NC + c
    c0 = _chunk0(wid)

    zv = jnp.zeros((16,), jnp.float32)
    iota = lax.iota(jnp.int32, 16)
    onev = jnp.where(iota == 0, 1.0, 0.0).astype(jnp.float32)

    _fill_zero(zbuf, ZR)

    def fill_ones(i, _):
        ones[i, pl.ds(0, 16)] = onev
        for j in range(1, D // 16):
            ones[i, pl.ds(j * 16, 16)] = zv
        return 0
    lax.fori_loop(0, CH, fill_ones, 0)

    _zero_acc(zbuf, acc, s)
    plsc.subcore_barrier()

    pltpu.sync_copy(dst3.at[c0], idx_d.at[0])

    def pair(k, _):
        pltpu.async_copy(ones, acc.at[idx_d.at[0, 0]], ss0, add=True)

        @pl.when(k > 0)
        def _():
            pltpu.make_async_copy(ones, acc.at[idx_d.at[1, 0]], ss1).wait()
        pltpu.sync_copy(dst3.at[_clamp(c0 + 2 * k + 1)], idx_d.at[1])
        pltpu.async_copy(ones, acc.at[idx_d.at[1, 0]], ss1, add=True)
        pltpu.make_async_copy(ones, acc.at[idx_d.at[0, 0]], ss0).wait()
        pltpu.sync_copy(dst3.at[_clamp(c0 + 2 * k + 2)], idx_d.at[0])
        return 0
    lax.fori_loop(0, PAIRS, pair, 0)

    pltpu.make_async_copy(ones, acc.at[idx_d.at[1, 0]], ss1).wait()

    @pl.when(wid < NEXTRA)
    def _last():
        pltpu.sync_copy(ones, acc.at[idx_d.at[0, 0]], add=True)
    plsc.subcore_barrier()

    _writeback(acc, deg_hbm, c, s)


ROWS_BLK = 1000


def _tc_body(apply_relu, p_ref, dg_ref, x_ref, wl_ref, wr_ref, b_ref, o_ref):
    agg = p_ref[0] + p_ref[1]
    deg = dg_ref[0, :, 0:1] + dg_ref[1, :, 0:1]
    mean = agg / jnp.maximum(deg, 1.0)
    h = jnp.dot(mean, wl_ref[...], preferred_element_type=jnp.float32)
    h = h + jnp.dot(x_ref[...], wr_ref[...], preferred_element_type=jnp.float32)
    h = h + b_ref[...]
    if apply_relu:
        h = jnp.maximum(h, 0.0)
    o_ref[...] = h


def _tc_combine(p, dg, x, wl, wr, b, apply_relu):
    grid = (N_NODES // ROWS_BLK,)
    return pl.pallas_call(
        functools.partial(_tc_body, apply_relu),
        grid=grid,
        in_specs=[
            pl.BlockSpec((NC, ROWS_BLK, D), lambda i: (0, i, 0)),
            pl.BlockSpec((NC, ROWS_BLK, D), lambda i: (0, i, 0)),
            pl.BlockSpec((ROWS_BLK, D), lambda i: (i, 0)),
            pl.BlockSpec((D, D), lambda i: (0, 0)),
            pl.BlockSpec((D, D), lambda i: (0, 0)),
            pl.BlockSpec((1, D), lambda i: (0, 0)),
        ],
        out_specs=pl.BlockSpec((ROWS_BLK, D), lambda i: (i, 0)),
        out_shape=jax.ShapeDtypeStruct((N_NODES, D), jnp.float32),
    )(p, dg, x, wl, wr, b)


def kernel(x, edge_index, edge_attr, W1_l, W1_r, b1, W2_l, W2_r, b2):
    src3 = edge_index[0].astype(jnp.int32).reshape(NCHUNK, 1, CH)
    dst3 = edge_index[1].astype(jnp.int32).reshape(NCHUNK, 1, CH)
    agg1 = _sc_agg(x, src3, dst3)
    degp = _sc_deg(dst3)
    h = _tc_combine(agg1, degp, x, W1_l, W1_r, b1.reshape(1, D), True)
    agg2 = _sc_agg(h, src3, dst3)
    out = _tc_combine(agg2, degp, h, W2_l, W2_r, b2.reshape(1, D), False)
    return out
